# baseline (speedup 1.0000x reference)
import numpy as np
import jax
import jax.numpy as jnp
from jax.experimental import pallas as pl
from jax.experimental.pallas import tpu as pltpu

R = 0.2
M = 5
TABLE = 2.0


def _mod2(h):
    return h - TABLE * jnp.floor(h / TABLE)


def _bucket_k(WfT, a0T, am, bh):
    sq = jnp.sum(WfT * WfT, axis=0, keepdims=True)
    norms = jnp.sqrt(sq)
    scale = 0.75 / jnp.maximum(jnp.max(norms), 1e-12)
    ksdot = jnp.sum(WfT * a0T, axis=0, keepdims=True) * scale
    v = norms * scale
    pdot = jnp.zeros_like(ksdot)
    for j in range(M):
        v = v * v
        pdot = pdot + v * am[0, j]
    hk = jnp.floor((ksdot + pdot + bh) / R)
    return _mod2(hk)


def _bucket_q(Sp, Ysel, Xsel, Amat, am, bh, denom):
    Vs = jnp.dot(Ysel, Sp, preferred_element_type=jnp.float32,
                 precision=jax.lax.Precision.HIGHEST)
    Wm = jnp.dot(Vs, Xsel, preferred_element_type=jnp.float32,
                 precision=jax.lax.Precision.HIGHEST)
    s1 = jnp.sum(Wm * Amat)
    s2 = jnp.sum(Wm * Wm)
    t = s1 / denom
    nrm = jnp.sqrt(s2) / denom
    hq = jnp.floor((t / jnp.maximum(nrm, 1e-12)
                    + 0.5 * jnp.sum(am) + bh[0, 0]) / R)
    return _mod2(hq)


def _mask(bk, bq):
    m = (bk == bq).astype(jnp.float32)
    return jnp.where(jnp.sum(m) == 0.0, jnp.ones_like(m), m)


def _conv(xb, Bcat, H, precision):
    Hp, Bc, Wcp = xb.shape
    WOp = Bcat.shape[1]
    Xcat = jnp.concatenate([xb[dy:dy + H] for dy in range(5)], axis=2)
    P = jnp.dot(Xcat.reshape(H * Bc, 5 * Wcp), Bcat,
                preferred_element_type=jnp.float32, precision=precision)
    return P.reshape(H, Bc, WOp)


def _pool(acc, half):
    H, Bc, _ = acc.shape
    px = jnp.maximum(acc[:, :, :half], acc[:, :, half:])
    return jnp.max(px.reshape(H // 2, 2, Bc, half), axis=1)


def _layer1_body(x_ref, Bf_ref, bt_ref, y_ref, s1_ref, s0_ref):
    i = pl.program_id(0)
    xb = x_ref[...]
    acc = _conv(xb, Bf_ref[...], 32,
                jax.lax.Precision.HIGHEST)
    acc = jnp.maximum(acc + bt_ref[...].reshape(1, 1, -1), 0.0)
    p = _pool(acc, 384)
    yp = jnp.pad(p, ((2, 2), (0, 0), (0, 0)))
    y_ref[...] = yp
    s1p = jnp.sum(yp, axis=1)
    s0p = jnp.sum(xb, axis=1)

    @pl.when(i == 0)
    def _():
        s1_ref[...] = s1p
        s0_ref[...] = s0p

    @pl.when(i > 0)
    def _():
        s1_ref[...] += s1p
        s0_ref[...] += s0p


def _layer2_body(x_ref, Bf_ref, bt_ref, s0_ref, s1_ref,
                 w1t_ref, a01_ref, am1_ref, bh1_ref,
                 ys1_ref, xs1_ref, amat1_ref, tin_ref,
                 w2t_ref, a02_ref, am2_ref, bh2_ref,
                 ys2_ref, xs2_ref, amat2_ref, tout_ref,
                 y_ref, s2_ref, m1v_ref, m2v_ref):
    i = pl.program_id(0)
    B = pl.num_programs(0) * x_ref.shape[1]

    @pl.when(i == 0)
    def _():
        bq1 = _bucket_q(s0_ref[...], ys1_ref[...], xs1_ref[...],
                        amat1_ref[...], am1_ref[...], bh1_ref[...],
                        float(B) * 32.0 * 32.0)
        bk1 = _bucket_k(w1t_ref[...], a01_ref[...], am1_ref[...],
                        bh1_ref[...])
        m1 = _mask(bk1, bq1)
        m1v = jnp.dot(m1, tin_ref[...],
                      preferred_element_type=jnp.float32)
        m1v_ref[...] = m1v
        s1m = s1_ref[...] * m1v
        bq2 = _bucket_q(s1m, ys2_ref[...], xs2_ref[...],
                        amat2_ref[...], am2_ref[...], bh2_ref[...],
                        float(B) * 16.0 * 16.0)
        bk2 = _bucket_k(w2t_ref[...], a02_ref[...], am2_ref[...],
                        bh2_ref[...])
        m2 = _mask(bk2, bq2)
        m2v_ref[...] = jnp.dot(m2, tout_ref[...],
                               preferred_element_type=jnp.float32)

    xb = x_ref[...] * m1v_ref[...].reshape(1, 1, -1)
    acc = _conv(xb, Bf_ref[...], 16,
                jax.lax.Precision.HIGHEST)
    acc = (acc + bt_ref[...].reshape(1, 1, -1)) * m2v_ref[...].reshape(1, 1, -1)
    acc = jnp.maximum(acc, 0.0)
    p = _pool(acc, 256)
    yp = jnp.pad(p, ((2, 2), (0, 0), (0, 0)))
    y_ref[...] = yp
    s2p = jnp.sum(yp, axis=1)

    @pl.when(i == 0)
    def _():
        s2_ref[...] = s2p

    @pl.when(i > 0)
    def _():
        s2_ref[...] += s2p


def _layer3_body(x_ref, Bf_ref, bt_ref, s2_ref,
                 w3t_ref, a03_ref, am3_ref, bh3_ref,
                 ys3_ref, xs3_ref, amat3_ref, tout_ref,
                 wout_ref, bout_ref, o_ref, m3v_ref):
    i = pl.program_id(0)
    B = pl.num_programs(0) * x_ref.shape[1]

    @pl.when(i == 0)
    def _():
        bq3 = _bucket_q(s2_ref[...], ys3_ref[...], xs3_ref[...],
                        amat3_ref[...], am3_ref[...], bh3_ref[...],
                        float(B) * 8.0 * 8.0)
        bk3 = _bucket_k(w3t_ref[...], a03_ref[...], am3_ref[...],
                        bh3_ref[...])
        m3 = _mask(bk3, bq3)
        m3v_ref[...] = jnp.dot(m3, tout_ref[...],
                               preferred_element_type=jnp.float32)

    xb = x_ref[...]
    acc = _conv(xb, Bf_ref[...], 8,
                jax.lax.Precision.HIGHEST)
    acc = (acc + bt_ref[...].reshape(1, 1, -1)) * m3v_ref[...].reshape(1, 1, -1)
    acc = jnp.maximum(acc, 0.0)
    p = _pool(acc, 128)
    out = bout_ref[...]
    for y in range(4):
        out = out + jnp.dot(p[y], wout_ref[y],
                            preferred_element_type=jnp.float32)
    o_ref[...] = out


def _colmap(O, Wsp, half, lpad):
    x = np.arange(Wsp)[:, None]
    o = np.arange(O)[None, :]
    return ((x % 2) * half + lpad + (x // 2) * O + o).reshape(-1)


def _bfull(Wl, Wsp, Wcp, half, lpad):
    O, C, _, _ = Wl.shape
    eye = jnp.eye(Wsp, dtype=jnp.float32)
    rows = []
    for dy in range(5):
        Bdy = jnp.zeros(((Wsp + 4) * C, Wsp * O), jnp.float32)
        for dx in range(5):
            Wt = Wl[:, :, dy, dx].T
            blk = (eye[:, None, :, None] * Wt[None, :, None, :]
                   ).reshape(Wsp * C, Wsp * O)
            Bdy = Bdy + jnp.pad(blk, ((dx * C, (4 - dx) * C), (0, 0)))
        rows.append(jnp.pad(Bdy, ((0, Wcp - (Wsp + 4) * C), (0, 0))))
    Bold = jnp.concatenate(rows, axis=0)
    cm = _colmap(O, Wsp, half, lpad)
    return jnp.zeros((5 * Wcp, 2 * half), jnp.float32).at[:, cm].set(Bold)


def _btile(b, Wsp, half, lpad):
    cm = _colmap(b.shape[0], Wsp, half, lpad)
    return jnp.zeros((1, 2 * half), jnp.float32).at[0, cm].set(
        jnp.tile(b, (Wsp,)))


def _touttile(O, Wsp, half, lpad):
    cm = _colmap(O, Wsp, half, lpad)
    told = jnp.tile(jnp.eye(O, dtype=jnp.float32), (1, Wsp))
    return jnp.zeros((O, 2 * half), jnp.float32).at[:, cm].set(told)


def _tintile(C, Wp, Wcp):
    t = jnp.tile(jnp.eye(C, dtype=jnp.float32), (1, Wp))
    return jnp.pad(t, ((0, 0), (0, Wcp - Wp * C)))


def _selmats(Hp, H, Wsp, C, Wcp):
    ys = np.zeros((8, Hp), np.float32)
    for dy in range(5):
        ys[dy, dy:dy + H] = 1.0
    Wp = Wsp + 4
    xi = np.arange(Wp)[:, None, None, None]
    ci = np.arange(C)[None, :, None, None]
    dxi = np.arange(5)[None, None, :, None]
    cj = np.arange(C)[None, None, None, :]
    xs = ((ci == cj) & (xi - dxi >= 0) & (xi - dxi < Wsp)).astype(np.float32)
    xs = xs.reshape(Wp * C, 5 * C)
    xs = np.pad(xs, ((0, Wcp - Wp * C), (0, 0)))
    return jnp.asarray(ys), jnp.asarray(xs)


def _amat(a, C):
    aq = a[:C * 25].reshape(C, 5, 5)
    am8 = jnp.zeros((8, 5 * C), jnp.float32)
    return am8.at[:5].set(jnp.transpose(aq, (1, 2, 0)).reshape(5, 5 * C))


def _fspec(shape):
    n = len(shape)
    return pl.BlockSpec(shape, lambda i, n=n: (0,) * n)


def kernel(x, W1, b1, W2, b2, W3, b3, Wout, bout, a1, bh1, a2, bh2, a3, bh3):
    B = x.shape[0]
    Bc = 64 if B % 64 == 0 else B
    grid = B // Bc
    f32 = jnp.float32

    xq = jnp.transpose(x, (2, 0, 3, 1)).reshape(32, B, 96)
    xq = jnp.pad(xq, ((2, 2), (0, 0), (6, 26)))

    Bf1 = _bfull(W1, 32, 128, 384, 32)
    Bf2 = _bfull(W2, 16, 384, 256, 40)
    Bf3 = _bfull(W3, 8, 256, 128, 0)
    bt1 = _btile(b1, 32, 384, 32)
    bt2 = _btile(b2, 16, 256, 40)
    bt3 = _btile(b3, 8, 128, 0)

    ys1, xs1 = _selmats(36, 32, 32, 3, 128)
    ys2, xs2 = _selmats(20, 16, 16, 16, 384)
    ys3, xs3 = _selmats(12, 8, 8, 20, 256)
    amat1, amat2, amat3 = _amat(a1, 3), _amat(a2, 16), _amat(a3, 20)
    w1t = W1.reshape(16, 75).T
    w2t = W2.reshape(20, 400).T
    w3t = W3.reshape(20, 500).T
    a01 = a1[:75].reshape(75, 1)
    a02 = a2[:400].reshape(400, 1)
    a03 = a3[:500].reshape(500, 1)
    am1 = a1[75:].reshape(1, M)
    am2 = a2[400:].reshape(1, M)
    am3 = a3[500:].reshape(1, M)
    bh1r = bh1.reshape(1, 1)
    bh2r = bh2.reshape(1, 1)
    bh3r = bh3.reshape(1, 1)
    tin2 = _tintile(16, 20, 384)
    tout2 = _touttile(20, 16, 256, 40)
    tout3 = _touttile(20, 8, 128, 0)
    Wr = jnp.transpose(Wout.reshape(10, 20, 4, 4), (2, 3, 1, 0))
    W4 = jnp.zeros((4, 128, 10), f32).at[:, :80, :].set(Wr.reshape(4, 80, 10))
    boutr = bout.reshape(1, 10)

    xspec = lambda hp, wc: pl.BlockSpec((hp, Bc, wc), lambda i: (0, i, 0))

    y1p, s1raw, s0 = pl.pallas_call(
        _layer1_body,
        grid=(grid,),
        in_specs=[xspec(36, 128), _fspec((640, 768)), _fspec((1, 768))],
        out_specs=[pl.BlockSpec((20, Bc, 384), lambda i: (0, i, 0)),
                   _fspec((20, 384)), _fspec((36, 128))],
        out_shape=[jax.ShapeDtypeStruct((20, B, 384), f32),
                   jax.ShapeDtypeStruct((20, 384), f32),
                   jax.ShapeDtypeStruct((36, 128), f32)],
    )(xq, Bf1, bt1)

    y2p, s2 = pl.pallas_call(
        _layer2_body,
        grid=(grid,),
        in_specs=[xspec(20, 384), _fspec((1920, 512)), _fspec((1, 512)),
                  _fspec((36, 128)), _fspec((20, 384)),
                  _fspec((75, 16)), _fspec((75, 1)), _fspec((1, M)),
                  _fspec((1, 1)),
                  _fspec((8, 36)), _fspec((128, 15)), _fspec((8, 15)),
                  _fspec((16, 384)),
                  _fspec((400, 20)), _fspec((400, 1)), _fspec((1, M)),
                  _fspec((1, 1)),
                  _fspec((8, 20)), _fspec((384, 80)), _fspec((8, 80)),
                  _fspec((20, 512))],
        out_specs=[pl.BlockSpec((12, Bc, 256), lambda i: (0, i, 0)),
                   _fspec((12, 256))],
        out_shape=[jax.ShapeDtypeStruct((12, B, 256), f32),
                   jax.ShapeDtypeStruct((12, 256), f32)],
        scratch_shapes=[pltpu.VMEM((1, 384), f32), pltpu.VMEM((1, 512), f32)],
    )(y1p, Bf2, bt2, s0, s1raw,
      w1t, a01, am1, bh1r, ys1, xs1, amat1, tin2,
      w2t, a02, am2, bh2r, ys2, xs2, amat2, tout2)

    out = pl.pallas_call(
        _layer3_body,
        grid=(grid,),
        in_specs=[xspec(12, 256), _fspec((1280, 256)), _fspec((1, 256)),
                  _fspec((12, 256)),
                  _fspec((500, 20)), _fspec((500, 1)), _fspec((1, M)),
                  _fspec((1, 1)),
                  _fspec((8, 12)), _fspec((256, 100)), _fspec((8, 100)),
                  _fspec((20, 256)),
                  _fspec((4, 128, 10)), _fspec((1, 10))],
        out_specs=pl.BlockSpec((Bc, 10), lambda i: (i, 0)),
        out_shape=jax.ShapeDtypeStruct((B, 10), f32),
        scratch_shapes=[pltpu.VMEM((1, 256), f32)],
    )(y2p, Bf3, bt3, s2,
      w3t, a03, am3, bh3r, ys3, xs3, amat3, tout3, W4, boutr)

    return out


# speedup vs baseline: 1.9810x; 1.9810x over previous
import numpy as np
import jax
import jax.numpy as jnp
from jax.experimental import pallas as pl
from jax.experimental.pallas import tpu as pltpu

R = 0.2
M = 5
TABLE = 2.0


def _mod2(h):
    return h - TABLE * jnp.floor(h / TABLE)


_HI = jax.lax.Precision.HIGHEST


def _bucket_k(WfT, a0r, am, bh):
    D0 = WfT.shape[0]
    ones = jnp.ones((1, D0), jnp.float32)
    sq = jnp.dot(ones, WfT * WfT, preferred_element_type=jnp.float32,
                 precision=_HI)
    norms = jnp.sqrt(sq)
    scale = 0.75 / jnp.maximum(jnp.max(norms), 1e-12)
    ksdot = jnp.dot(a0r, WfT, preferred_element_type=jnp.float32,
                    precision=_HI) * scale
    v = norms * scale
    pdot = jnp.zeros_like(ksdot)
    for j in range(M):
        v = v * v
        pdot = pdot + v * am[0, j]
    hk = jnp.floor((ksdot + pdot + bh) / R)
    return _mod2(hk)


def _bucket_q(Sp, Ysel, Xsel, Amat, am, bh, denom):
    Vs = jnp.dot(Ysel, Sp, preferred_element_type=jnp.float32,
                 precision=_HI)
    Wm = jnp.dot(Vs, Xsel, preferred_element_type=jnp.float32,
                 precision=_HI)
    ones8 = jnp.ones((1, 8), jnp.float32)
    onesc = jnp.ones((Wm.shape[1], 1), jnp.float32)
    s1 = jnp.dot(jnp.dot(ones8, Wm * Amat, precision=_HI), onesc,
                 precision=_HI)[0, 0]
    s2 = jnp.dot(jnp.dot(ones8, Wm * Wm, precision=_HI), onesc,
                 precision=_HI)[0, 0]
    t = s1 / denom
    nrm = jnp.sqrt(s2) / denom
    hq = jnp.floor((t / jnp.maximum(nrm, 1e-12)
                    + 0.5 * jnp.sum(am) + bh[0, 0]) / R)
    return _mod2(hq)


def _mask(bk, bq):
    m = (bk == bq).astype(jnp.float32)
    return jnp.where(jnp.sum(m) == 0.0, jnp.ones_like(m), m)


def _conv(xb, Bcat, H, precision):
    Hp, Bc, Wcp = xb.shape
    WOp = Bcat.shape[1]
    Xcat = jnp.concatenate([xb[dy:dy + H] for dy in range(5)], axis=2)
    P = jnp.dot(Xcat.reshape(H * Bc, 5 * Wcp), Bcat,
                preferred_element_type=jnp.float32, precision=precision)
    return P.reshape(H, Bc, WOp)


def _pool(acc, half):
    H, Bc, _ = acc.shape
    px = jnp.maximum(acc[:, :, :half], acc[:, :, half:])
    return jnp.max(px.reshape(H // 2, 2, Bc, half), axis=1)


def _layer1_body(x_ref, Bf_ref, bt_ref, y_ref, s1_ref, s0_ref):
    i = pl.program_id(0)
    xb = x_ref[...]
    acc = _conv(xb, Bf_ref[...], 32,
                jax.lax.Precision.DEFAULT)
    acc = jnp.maximum(acc + bt_ref[...].reshape(1, 1, -1), 0.0)
    p = _pool(acc, 384)
    yp = jnp.pad(p, ((2, 2), (0, 0), (0, 0)))
    y_ref[...] = yp
    s1p = jnp.sum(yp, axis=1)
    s0p = jnp.sum(xb, axis=1)

    @pl.when(i == 0)
    def _():
        s1_ref[...] = s1p
        s0_ref[...] = s0p

    @pl.when(i > 0)
    def _():
        s1_ref[...] += s1p
        s0_ref[...] += s0p


def _layer2_body(x_ref, Bf_ref, bt_ref, s0_ref, s1_ref,
                 w1t_ref, a01_ref, am1_ref, bh1_ref,
                 ys1_ref, xs1_ref, amat1_ref, tin_ref,
                 w2t_ref, a02_ref, am2_ref, bh2_ref,
                 ys2_ref, xs2_ref, amat2_ref, tout_ref,
                 y_ref, s2_ref, m1v_ref, m2v_ref):
    i = pl.program_id(0)
    B = pl.num_programs(0) * x_ref.shape[1]

    @pl.when(i == 0)
    def _():
        bq1 = _bucket_q(s0_ref[...], ys1_ref[...], xs1_ref[...],
                        amat1_ref[...], am1_ref[...], bh1_ref[...],
                        float(B) * 32.0 * 32.0)
        bk1 = _bucket_k(w1t_ref[...], a01_ref[...], am1_ref[...],
                        bh1_ref[...])
        m1 = _mask(bk1, bq1)
        m1v = jnp.dot(m1, tin_ref[...],
                      preferred_element_type=jnp.float32)
        m1v_ref[...] = m1v
        s1m = s1_ref[...] * m1v
        bq2 = _bucket_q(s1m, ys2_ref[...], xs2_ref[...],
                        amat2_ref[...], am2_ref[...], bh2_ref[...],
                        float(B) * 16.0 * 16.0)
        bk2 = _bucket_k(w2t_ref[...], a02_ref[...], am2_ref[...],
                        bh2_ref[...])
        m2 = _mask(bk2, bq2)
        m2v_ref[...] = jnp.dot(m2, tout_ref[...],
                               preferred_element_type=jnp.float32)

    xb = x_ref[...] * m1v_ref[...].reshape(1, 1, -1)
    acc = _conv(xb, Bf_ref[...], 16,
                jax.lax.Precision.DEFAULT)
    acc = (acc + bt_ref[...].reshape(1, 1, -1)) * m2v_ref[...].reshape(1, 1, -1)
    acc = jnp.maximum(acc, 0.0)
    p = _pool(acc, 256)
    yp = jnp.pad(p, ((2, 2), (0, 0), (0, 0)))
    y_ref[...] = yp
    s2p = jnp.sum(yp, axis=1)

    @pl.when(i == 0)
    def _():
        s2_ref[...] = s2p

    @pl.when(i > 0)
    def _():
        s2_ref[...] += s2p


def _layer3_body(x_ref, Bf_ref, bt_ref, s2_ref,
                 w3t_ref, a03_ref, am3_ref, bh3_ref,
                 ys3_ref, xs3_ref, amat3_ref, tout_ref,
                 wout_ref, bout_ref, o_ref, m3v_ref):
    i = pl.program_id(0)
    B = pl.num_programs(0) * x_ref.shape[1]

    @pl.when(i == 0)
    def _():
        bq3 = _bucket_q(s2_ref[...], ys3_ref[...], xs3_ref[...],
                        amat3_ref[...], am3_ref[...], bh3_ref[...],
                        float(B) * 8.0 * 8.0)
        bk3 = _bucket_k(w3t_ref[...], a03_ref[...], am3_ref[...],
                        bh3_ref[...])
        m3 = _mask(bk3, bq3)
        m3v_ref[...] = jnp.dot(m3, tout_ref[...],
                               preferred_element_type=jnp.float32)

    xb = x_ref[...]
    acc = _conv(xb, Bf_ref[...], 8,
                jax.lax.Precision.DEFAULT)
    acc = (acc + bt_ref[...].reshape(1, 1, -1)) * m3v_ref[...].reshape(1, 1, -1)
    acc = jnp.maximum(acc, 0.0)
    p = _pool(acc, 128)
    out = bout_ref[...]
    for y in range(4):
        out = out + jnp.dot(p[y], wout_ref[y],
                            preferred_element_type=jnp.float32)
    o_ref[...] = out


def _colmap(O, Wsp, half, lpad):
    x = np.arange(Wsp)[:, None]
    o = np.arange(O)[None, :]
    return ((x % 2) * half + lpad + (x // 2) * O + o).reshape(-1)


def _bfull(Wl, Wsp, Wcp, half, lpad):
    O, C, _, _ = Wl.shape
    eye = jnp.eye(Wsp, dtype=jnp.float32)
    rows = []
    for dy in range(5):
        Bdy = jnp.zeros(((Wsp + 4) * C, Wsp * O), jnp.float32)
        for dx in range(5):
            Wt = Wl[:, :, dy, dx].T
            blk = (eye[:, None, :, None] * Wt[None, :, None, :]
                   ).reshape(Wsp * C, Wsp * O)
            Bdy = Bdy + jnp.pad(blk, ((dx * C, (4 - dx) * C), (0, 0)))
        rows.append(jnp.pad(Bdy, ((0, Wcp - (Wsp + 4) * C), (0, 0))))
    Bold = jnp.concatenate(rows, axis=0)
    cm = _colmap(O, Wsp, half, lpad)
    return jnp.zeros((5 * Wcp, 2 * half), jnp.float32).at[:, cm].set(Bold)


def _btile(b, Wsp, half, lpad):
    cm = _colmap(b.shape[0], Wsp, half, lpad)
    return jnp.zeros((1, 2 * half), jnp.float32).at[0, cm].set(
        jnp.tile(b, (Wsp,)))


def _touttile(O, Wsp, half, lpad):
    cm = _colmap(O, Wsp, half, lpad)
    told = jnp.tile(jnp.eye(O, dtype=jnp.float32), (1, Wsp))
    return jnp.zeros((O, 2 * half), jnp.float32).at[:, cm].set(told)


def _tintile(C, Wp, Wcp):
    t = jnp.tile(jnp.eye(C, dtype=jnp.float32), (1, Wp))
    return jnp.pad(t, ((0, 0), (0, Wcp - Wp * C)))


def _selmats(Hp, H, Wsp, C, Wcp):
    ys = np.zeros((8, Hp), np.float32)
    for dy in range(5):
        ys[dy, dy:dy + H] = 1.0
    Wp = Wsp + 4
    xi = np.arange(Wp)[:, None, None, None]
    ci = np.arange(C)[None, :, None, None]
    dxi = np.arange(5)[None, None, :, None]
    cj = np.arange(C)[None, None, None, :]
    xs = ((ci == cj) & (xi - dxi >= 0) & (xi - dxi < Wsp)).astype(np.float32)
    xs = xs.reshape(Wp * C, 5 * C)
    xs = np.pad(xs, ((0, Wcp - Wp * C), (0, 0)))
    return jnp.asarray(ys), jnp.asarray(xs)


def _amat(a, C):
    aq = a[:C * 25].reshape(C, 5, 5)
    am8 = jnp.zeros((8, 5 * C), jnp.float32)
    return am8.at[:5].set(jnp.transpose(aq, (1, 2, 0)).reshape(5, 5 * C))


def _fspec(shape):
    n = len(shape)
    return pl.BlockSpec(shape, lambda i, n=n: (0,) * n)


def kernel(x, W1, b1, W2, b2, W3, b3, Wout, bout, a1, bh1, a2, bh2, a3, bh3):
    B = x.shape[0]
    Bc = 64 if B % 64 == 0 else B
    grid = B // Bc
    f32 = jnp.float32

    xq = jnp.transpose(x, (2, 0, 3, 1)).reshape(32, B, 96)
    xq = jnp.pad(xq, ((2, 2), (0, 0), (6, 26)))

    Bf1 = _bfull(W1, 32, 128, 384, 32)
    Bf2 = _bfull(W2, 16, 384, 256, 40)
    Bf3 = _bfull(W3, 8, 256, 128, 0)
    bt1 = _btile(b1, 32, 384, 32)
    bt2 = _btile(b2, 16, 256, 40)
    bt3 = _btile(b3, 8, 128, 0)

    ys1, xs1 = _selmats(36, 32, 32, 3, 128)
    ys2, xs2 = _selmats(20, 16, 16, 16, 384)
    ys3, xs3 = _selmats(12, 8, 8, 20, 256)
    amat1, amat2, amat3 = _amat(a1, 3), _amat(a2, 16), _amat(a3, 20)
    w1t = W1.reshape(16, 75).T
    w2t = W2.reshape(20, 400).T
    w3t = W3.reshape(20, 500).T
    a01 = a1[:75].reshape(1, 75)
    a02 = a2[:400].reshape(1, 400)
    a03 = a3[:500].reshape(1, 500)
    am1 = a1[75:].reshape(1, M)
    am2 = a2[400:].reshape(1, M)
    am3 = a3[500:].reshape(1, M)
    bh1r = bh1.reshape(1, 1)
    bh2r = bh2.reshape(1, 1)
    bh3r = bh3.reshape(1, 1)
    tin2 = _tintile(16, 20, 384)
    tout2 = _touttile(20, 16, 256, 40)
    tout3 = _touttile(20, 8, 128, 0)
    Wr = jnp.transpose(Wout.reshape(10, 20, 4, 4), (2, 3, 1, 0))
    W4 = jnp.zeros((4, 128, 10), f32).at[:, :80, :].set(Wr.reshape(4, 80, 10))
    boutr = bout.reshape(1, 10)

    xspec = lambda hp, wc: pl.BlockSpec((hp, Bc, wc), lambda i: (0, i, 0))

    y1p, s1raw, s0 = pl.pallas_call(
        _layer1_body,
        grid=(grid,),
        in_specs=[xspec(36, 128), _fspec((640, 768)), _fspec((1, 768))],
        out_specs=[pl.BlockSpec((20, Bc, 384), lambda i: (0, i, 0)),
                   _fspec((20, 384)), _fspec((36, 128))],
        out_shape=[jax.ShapeDtypeStruct((20, B, 384), f32),
                   jax.ShapeDtypeStruct((20, 384), f32),
                   jax.ShapeDtypeStruct((36, 128), f32)],
    )(xq, Bf1, bt1)

    y2p, s2 = pl.pallas_call(
        _layer2_body,
        grid=(grid,),
        in_specs=[xspec(20, 384), _fspec((1920, 512)), _fspec((1, 512)),
                  _fspec((36, 128)), _fspec((20, 384)),
                  _fspec((75, 16)), _fspec((1, 75)), _fspec((1, M)),
                  _fspec((1, 1)),
                  _fspec((8, 36)), _fspec((128, 15)), _fspec((8, 15)),
                  _fspec((16, 384)),
                  _fspec((400, 20)), _fspec((1, 400)), _fspec((1, M)),
                  _fspec((1, 1)),
                  _fspec((8, 20)), _fspec((384, 80)), _fspec((8, 80)),
                  _fspec((20, 512))],
        out_specs=[pl.BlockSpec((12, Bc, 256), lambda i: (0, i, 0)),
                   _fspec((12, 256))],
        out_shape=[jax.ShapeDtypeStruct((12, B, 256), f32),
                   jax.ShapeDtypeStruct((12, 256), f32)],
        scratch_shapes=[pltpu.VMEM((1, 384), f32), pltpu.VMEM((1, 512), f32)],
    )(y1p, Bf2, bt2, s0, s1raw,
      w1t, a01, am1, bh1r, ys1, xs1, amat1, tin2,
      w2t, a02, am2, bh2r, ys2, xs2, amat2, tout2)

    out = pl.pallas_call(
        _layer3_body,
        grid=(grid,),
        in_specs=[xspec(12, 256), _fspec((1280, 256)), _fspec((1, 256)),
                  _fspec((12, 256)),
                  _fspec((500, 20)), _fspec((1, 500)), _fspec((1, M)),
                  _fspec((1, 1)),
                  _fspec((8, 12)), _fspec((256, 100)), _fspec((8, 100)),
                  _fspec((20, 256)),
                  _fspec((4, 128, 10)), _fspec((1, 10))],
        out_specs=pl.BlockSpec((Bc, 10), lambda i: (i, 0)),
        out_shape=jax.ShapeDtypeStruct((B, 10), f32),
        scratch_shapes=[pltpu.VMEM((1, 256), f32)],
    )(y2p, Bf3, bt3, s2,
      w3t, a03, am3, bh3r, ys3, xs3, amat3, tout3, W4, boutr)

    return out
